# baseline (speedup 1.0000x reference)
import functools

import jax
import jax.numpy as jnp
import numpy as np
from jax import lax
from jax.experimental import pallas as pl
from jax.experimental.pallas import tpu as pltpu
from jax.experimental.pallas import tpu_sc as plsc

_N = 320000
_D = 128
_L = 16
_NJ = _D // _L
_NC = 2
_NS = 16
_NW = _NC * _NS
_RPW = _N // _NW

_NS_SC = 102400
_RPW_A = _NS_SC // _NW

_CA = 200
_NCH_A = _RPW_A // _CA
_CB = 200
_NCH_B = _RPW // _CB

_mesh = plsc.VectorSubcoreMesh(
    core_axis_name="c", subcore_axis_name="s", num_cores=_NC, num_subcores=_NS
)


def _worker_id():
    return lax.axis_index("s") * _NC + lax.axis_index("c")


@functools.partial(
    pl.kernel,
    out_type=jax.ShapeDtypeStruct((_NW * _D,), jnp.float32),
    mesh=_mesh,
    scratch_types=[
        pltpu.VMEM((2, _CA, _D), jnp.float32),
        pltpu.VMEM((_D,), jnp.float32),
        pltpu.SemaphoreType.DMA,
        pltpu.SemaphoreType.DMA,
    ],
)
def _partial_sums(x_hbm, out_hbm, buf, accv, sem0, sem1):
    wid = _worker_id()
    base = wid * _RPW_A
    sems = (sem0, sem1)

    for b in range(2):
        pltpu.async_copy(
            x_hbm.at[pl.ds(base + b * _CA, _CA)], buf.at[b], sems[b]
        )

    zero = jnp.zeros((_L,), jnp.float32)
    init = (zero,) * _NJ

    @pl.loop(0, _NCH_A // 2, init_carry=init)
    def acc_loop(g, acc):
        for b in range(2):
            pltpu.make_async_copy(
                x_hbm.at[pl.ds(0, _CA)], buf.at[b], sems[b]
            ).wait()

            @pl.loop(0, _CA, init_carry=acc, unroll=4)
            def row_loop(r, a):
                return tuple(
                    a[j] + buf[b, r, pl.ds(j * _L, _L)] for j in range(_NJ)
                )

            acc = row_loop

            @pl.when(g + 1 < _NCH_A // 2)
            def _():
                nxt = base + (2 * (g + 1) + b) * _CA
                pltpu.async_copy(
                    x_hbm.at[pl.ds(nxt, _CA)], buf.at[b], sems[b]
                )

        return acc

    for j in range(_NJ):
        accv[pl.ds(j * _L, _L)] = acc_loop[j]
    pltpu.sync_copy(accv, out_hbm.at[pl.ds(wid * _D, _D)])


@functools.partial(
    pl.kernel,
    out_type=jax.ShapeDtypeStruct((_N, _D), jnp.float32),
    mesh=_mesh,
    scratch_types=[
        pltpu.VMEM((_NW * _D,), jnp.float32),
        pltpu.VMEM((2, _CB, _D), jnp.float32),
        pltpu.VMEM((2, _CB, _D), jnp.float32),
        pltpu.SemaphoreType.DMA,
        pltpu.SemaphoreType.DMA,
        pltpu.SemaphoreType.DMA,
        pltpu.SemaphoreType.DMA,
    ],
)
def _subtract_mean(x_hbm, sums_hbm, out_hbm, pbuf, ibuf, obuf,
                   isem0, isem1, osem0, osem1):
    wid = _worker_id()
    base = wid * _RPW
    isems = (isem0, isem1)
    osems = (osem0, osem1)

    pltpu.sync_copy(sums_hbm, pbuf)

    zero = jnp.zeros((_L,), jnp.float32)

    @pl.loop(0, _NW, init_carry=(zero,) * _NJ, unroll=4)
    def red_loop(w, m):
        return tuple(
            m[j] + pbuf[pl.ds(w * _D + j * _L, _L)] for j in range(_NJ)
        )

    scale = np.float32(1.0 / _N)
    mean = tuple(m * scale for m in red_loop)

    for b in range(2):
        pltpu.async_copy(
            x_hbm.at[pl.ds(base + b * _CB, _CB)], ibuf.at[b], isems[b]
        )

    @pl.loop(0, _NCH_B // 2)
    def chunk_loop(g):
        for b in range(2):
            pltpu.make_async_copy(
                x_hbm.at[pl.ds(0, _CB)], ibuf.at[b], isems[b]
            ).wait()

            @pl.when(g >= 1)
            def _():
                pltpu.make_async_copy(
                    obuf.at[b], out_hbm.at[pl.ds(0, _CB)], osems[b]
                ).wait()

            @pl.loop(0, _CB, unroll=2)
            def row_loop(r):
                for j in range(_NJ):
                    sl = pl.ds(j * _L, _L)
                    obuf[b, r, sl] = ibuf[b, r, sl] - mean[j]

            c = 2 * g + b
            pltpu.async_copy(
                obuf.at[b], out_hbm.at[pl.ds(base + c * _CB, _CB)], osems[b]
            )

            @pl.when(g + 1 < _NCH_B // 2)
            def _():
                nxt = base + (2 * (g + 1) + b) * _CB
                pltpu.async_copy(
                    x_hbm.at[pl.ds(nxt, _CB)], ibuf.at[b], isems[b]
                )

    for b in range(2):
        pltpu.make_async_copy(
            obuf.at[b], out_hbm.at[pl.ds(0, _CB)], osems[b]
        ).wait()


_BM = 8000
_BMS = 6400

_NBLK_TCSUM = (_N - _NS_SC) // _BMS
_BLK0_TCSUM = _NS_SC // _BMS


def _sum_tc_body(x_ref, o_ref):
    @pl.when(pl.program_id(0) == 0)
    def _():
        o_ref[...] = jnp.zeros_like(o_ref)

    o_ref[...] += jnp.sum(x_ref[...], axis=0, keepdims=True)


_partial_sum_tc = pl.pallas_call(
    _sum_tc_body,
    grid=(_NBLK_TCSUM,),
    in_specs=[pl.BlockSpec((_BMS, _D), lambda i: (_BLK0_TCSUM + i, 0))],
    out_specs=pl.BlockSpec((1, _D), lambda i: (0, 0)),
    out_shape=jax.ShapeDtypeStruct((1, _D), jnp.float32),
)


def _sub_tc_body(p_sc_ref, p_tc_ref, x_ref, o_ref):
    total = jnp.sum(p_sc_ref[...], axis=0, keepdims=True) + p_tc_ref[...]
    o_ref[...] = x_ref[...] - total * np.float32(1.0 / _N)


_subtract_tc = pl.pallas_call(
    _sub_tc_body,
    grid=(_N // _BM,),
    in_specs=[
        pl.BlockSpec((_NW, _D), lambda i: (0, 0)),
        pl.BlockSpec((1, _D), lambda i: (0, 0)),
        pl.BlockSpec((_BM, _D), lambda i: (i, 0)),
    ],
    out_specs=pl.BlockSpec((_BM, _D), lambda i: (i, 0)),
    out_shape=jax.ShapeDtypeStruct((_N, _D), jnp.float32),
)


@jax.jit
def kernel(x, index):
    del index
    sums_sc = _partial_sums(x)
    sums_tc = _partial_sum_tc(x)
    return _subtract_tc(sums_sc.reshape(_NW, _D), sums_tc, x)


# speedup vs baseline: 1.0472x; 1.0472x over previous
import functools

import jax
import jax.numpy as jnp
import numpy as np
from jax import lax
from jax.experimental import pallas as pl
from jax.experimental.pallas import tpu as pltpu
from jax.experimental.pallas import tpu_sc as plsc

_N = 320000
_D = 128
_L = 16
_NJ = _D // _L
_NC = 2
_NS = 16
_NW = _NC * _NS
_RPW = _N // _NW

_NS_SC = 153600
_RPW_A = _NS_SC // _NW

_CA = 200
_NCH_A = _RPW_A // _CA
_CB = 200
_NCH_B = _RPW // _CB

_mesh = plsc.VectorSubcoreMesh(
    core_axis_name="c", subcore_axis_name="s", num_cores=_NC, num_subcores=_NS
)


def _worker_id():
    return lax.axis_index("s") * _NC + lax.axis_index("c")


@functools.partial(
    pl.kernel,
    out_type=jax.ShapeDtypeStruct((_NW * _D,), jnp.float32),
    mesh=_mesh,
    scratch_types=[
        pltpu.VMEM((2, _CA, _D), jnp.float32),
        pltpu.VMEM((_D,), jnp.float32),
        pltpu.SemaphoreType.DMA,
        pltpu.SemaphoreType.DMA,
    ],
)
def _partial_sums(x_hbm, out_hbm, buf, accv, sem0, sem1):
    wid = _worker_id()
    base = wid * _RPW_A
    sems = (sem0, sem1)

    for b in range(2):
        pltpu.async_copy(
            x_hbm.at[pl.ds(base + b * _CA, _CA)], buf.at[b], sems[b]
        )

    zero = jnp.zeros((_L,), jnp.float32)
    init = (zero,) * _NJ

    @pl.loop(0, _NCH_A // 2, init_carry=init)
    def acc_loop(g, acc):
        for b in range(2):
            pltpu.make_async_copy(
                x_hbm.at[pl.ds(0, _CA)], buf.at[b], sems[b]
            ).wait()

            @pl.loop(0, _CA, init_carry=acc, unroll=4)
            def row_loop(r, a):
                return tuple(
                    a[j] + buf[b, r, pl.ds(j * _L, _L)] for j in range(_NJ)
                )

            acc = row_loop

            @pl.when(g + 1 < _NCH_A // 2)
            def _():
                nxt = base + (2 * (g + 1) + b) * _CA
                pltpu.async_copy(
                    x_hbm.at[pl.ds(nxt, _CA)], buf.at[b], sems[b]
                )

        return acc

    for j in range(_NJ):
        accv[pl.ds(j * _L, _L)] = acc_loop[j]
    pltpu.sync_copy(accv, out_hbm.at[pl.ds(wid * _D, _D)])


@functools.partial(
    pl.kernel,
    out_type=jax.ShapeDtypeStruct((_N, _D), jnp.float32),
    mesh=_mesh,
    scratch_types=[
        pltpu.VMEM((_NW * _D,), jnp.float32),
        pltpu.VMEM((2, _CB, _D), jnp.float32),
        pltpu.VMEM((2, _CB, _D), jnp.float32),
        pltpu.SemaphoreType.DMA,
        pltpu.SemaphoreType.DMA,
        pltpu.SemaphoreType.DMA,
        pltpu.SemaphoreType.DMA,
    ],
)
def _subtract_mean(x_hbm, sums_hbm, out_hbm, pbuf, ibuf, obuf,
                   isem0, isem1, osem0, osem1):
    wid = _worker_id()
    base = wid * _RPW
    isems = (isem0, isem1)
    osems = (osem0, osem1)

    pltpu.sync_copy(sums_hbm, pbuf)

    zero = jnp.zeros((_L,), jnp.float32)

    @pl.loop(0, _NW, init_carry=(zero,) * _NJ, unroll=4)
    def red_loop(w, m):
        return tuple(
            m[j] + pbuf[pl.ds(w * _D + j * _L, _L)] for j in range(_NJ)
        )

    scale = np.float32(1.0 / _N)
    mean = tuple(m * scale for m in red_loop)

    for b in range(2):
        pltpu.async_copy(
            x_hbm.at[pl.ds(base + b * _CB, _CB)], ibuf.at[b], isems[b]
        )

    @pl.loop(0, _NCH_B // 2)
    def chunk_loop(g):
        for b in range(2):
            pltpu.make_async_copy(
                x_hbm.at[pl.ds(0, _CB)], ibuf.at[b], isems[b]
            ).wait()

            @pl.when(g >= 1)
            def _():
                pltpu.make_async_copy(
                    obuf.at[b], out_hbm.at[pl.ds(0, _CB)], osems[b]
                ).wait()

            @pl.loop(0, _CB, unroll=2)
            def row_loop(r):
                for j in range(_NJ):
                    sl = pl.ds(j * _L, _L)
                    obuf[b, r, sl] = ibuf[b, r, sl] - mean[j]

            c = 2 * g + b
            pltpu.async_copy(
                obuf.at[b], out_hbm.at[pl.ds(base + c * _CB, _CB)], osems[b]
            )

            @pl.when(g + 1 < _NCH_B // 2)
            def _():
                nxt = base + (2 * (g + 1) + b) * _CB
                pltpu.async_copy(
                    x_hbm.at[pl.ds(nxt, _CB)], ibuf.at[b], isems[b]
                )

    for b in range(2):
        pltpu.make_async_copy(
            obuf.at[b], out_hbm.at[pl.ds(0, _CB)], osems[b]
        ).wait()


_BM = 16000
_BMS = 6400

_NBLK_TCSUM = (_N - _NS_SC) // _BMS
_BLK0_TCSUM = _NS_SC // _BMS


def _sum_tc_body(x_ref, o_ref):
    @pl.when(pl.program_id(0) == 0)
    def _():
        o_ref[...] = jnp.zeros_like(o_ref)

    o_ref[...] += jnp.sum(x_ref[...], axis=0, keepdims=True)


_partial_sum_tc = pl.pallas_call(
    _sum_tc_body,
    grid=(_NBLK_TCSUM,),
    in_specs=[pl.BlockSpec((_BMS, _D), lambda i: (_BLK0_TCSUM + i, 0))],
    out_specs=pl.BlockSpec((1, _D), lambda i: (0, 0)),
    out_shape=jax.ShapeDtypeStruct((1, _D), jnp.float32),
)


def _sub_tc_body(p_sc_ref, p_tc_ref, x_ref, o_ref):
    total = jnp.sum(p_sc_ref[...], axis=0, keepdims=True) + p_tc_ref[...]
    o_ref[...] = x_ref[...] - total * np.float32(1.0 / _N)


_subtract_tc = pl.pallas_call(
    _sub_tc_body,
    grid=(_N // _BM,),
    in_specs=[
        pl.BlockSpec((_NW, _D), lambda i: (0, 0)),
        pl.BlockSpec((1, _D), lambda i: (0, 0)),
        pl.BlockSpec((_BM, _D), lambda i: (i, 0)),
    ],
    out_specs=pl.BlockSpec((_BM, _D), lambda i: (i, 0)),
    out_shape=jax.ShapeDtypeStruct((_N, _D), jnp.float32),
)


@jax.jit
def kernel(x, index):
    del index
    sums_sc = _partial_sums(x)
    sums_tc = _partial_sum_tc(x)
    return _subtract_tc(sums_sc.reshape(_NW, _D), sums_tc, x)
